# baseline (speedup 1.0000x reference)
import functools

import jax
import jax.numpy as jnp
from jax import lax
from jax.experimental import pallas as pl
from jax.experimental.pallas import tpu as pltpu
from jax.experimental.pallas import tpu_sc as plsc

VOCAB = 1000000
DIM = 64
MAX_LEN = 200
B = 4096
EPS = 1e-5

NC = 2
NS = 16
NW = NC * NS

NVREG = DIM // 16

SENT_PER_W = B // NW
SC_CHUNK = 4
TOK_CHUNK = SC_CHUNK * MAX_LEN
N_CHUNKS = SENT_PER_W // SC_CHUNK
GATHER_SUB = 128


def _rsqrt_scalar(x):
    i = lax.bitcast_convert_type(x, jnp.int32)
    i = jnp.int32(0x5F3759DF) - lax.shift_right_arithmetic(i, 1)
    y = lax.bitcast_convert_type(i, jnp.float32)
    for _ in range(3):
        y = y * (jnp.float32(1.5) - jnp.float32(0.5) * x * y * y)
    return y


def _make_kernel():
    mesh = plsc.VectorSubcoreMesh(core_axis_name="c", subcore_axis_name="s")

    @functools.partial(
        pl.kernel,
        out_type=jax.ShapeDtypeStruct((B * MAX_LEN, DIM), jnp.float32),
        mesh=mesh,
        scratch_types=[
            pltpu.VMEM((TOK_CHUNK,), jnp.int32),
            pltpu.VMEM((TOK_CHUNK, DIM), jnp.float32),
            pltpu.VMEM((MAX_LEN, DIM), jnp.float32),
            pltpu.VMEM((DIM,), jnp.float32),
            pltpu.VMEM((DIM,), jnp.float32),
            pltpu.SemaphoreType.DMA,
        ],
        compiler_params=pltpu.CompilerParams(
            needs_layout_passes=False, use_tc_tiling_on_sc=False),
    )
    def emb_kernel(ids_hbm, table_hbm, pos_hbm, scale_hbm, bias_hbm,
                   out_hbm, idx_v, rows_v, pos_v, scale_v, bias_v, sem):
        wid = lax.axis_index("s") * NC + lax.axis_index("c")
        w_base = wid * (SENT_PER_W * MAX_LEN)

        pltpu.sync_copy(pos_hbm, pos_v)
        pltpu.sync_copy(scale_hbm, scale_v)
        pltpu.sync_copy(bias_hbm, bias_v)

        sc = [scale_v[pl.ds(k * 16, 16)] for k in range(NVREG)]
        bs = [bias_v[pl.ds(k * 16, 16)] for k in range(NVREG)]

        def chunk_body(c, carry):
            tok_base = w_base + c * TOK_CHUNK
            pltpu.sync_copy(ids_hbm.at[pl.ds(tok_base, TOK_CHUNK)], idx_v)
            descs = []
            for g in range(0, TOK_CHUNK, GATHER_SUB):
                n = min(GATHER_SUB, TOK_CHUNK - g)
                descs.append(pltpu.async_copy(
                    table_hbm.at[idx_v.at[pl.ds(g, n)]],
                    rows_v.at[pl.ds(g, n)], sem))
            for d in descs:
                d.wait()

            def pos_body(p, pcarry):
                pv = [pos_v[p, pl.ds(k * 16, 16)] for k in range(NVREG)]
                for s in range(SC_CHUNK):
                    t = s * MAX_LEN + p
                    h = [rows_v[t, pl.ds(k * 16, 16)] + pv[k]
                         for k in range(NVREG)]
                    ssum = (h[0] + h[1]) + (h[2] + h[3])
                    qsum = h[0] * h[0]
                    for k in range(1, NVREG):
                        qsum = h[k] * h[k] + qsum
                    tot_s = plsc.cumsum(ssum)[15]
                    tot_q = plsc.cumsum(qsum)[15]
                    mean = tot_s * jnp.float32(1.0 / DIM)
                    var = tot_q * jnp.float32(1.0 / DIM) - mean * mean
                    rstd = _rsqrt_scalar(var + jnp.float32(EPS))
                    a = rstd
                    b = -mean * rstd
                    for k in range(NVREG):
                        rows_v[t, pl.ds(k * 16, 16)] = (h[k] * a + b) * sc[k] + bs[k]
                return pcarry

            lax.fori_loop(0, MAX_LEN, pos_body, 0)
            pltpu.sync_copy(rows_v, out_hbm.at[pl.ds(tok_base, TOK_CHUNK)])
            return carry

        lax.fori_loop(0, N_CHUNKS, chunk_body, 0)

    return emb_kernel


_EMB_KERNEL_CACHE = []


def kernel(input_ids, attention_mask, sentence_lengths, word_table,
           pos_table, ln_scale, ln_bias):
    del attention_mask, sentence_lengths
    if not _EMB_KERNEL_CACHE:
        _EMB_KERNEL_CACHE.append(_make_kernel())
    ids_flat = input_ids.reshape(B * MAX_LEN)
    out = _EMB_KERNEL_CACHE[0](ids_flat, word_table, pos_table, ln_scale,
                               ln_bias)
    return out.reshape(B, MAX_LEN, DIM)


# speedup vs baseline: 1.2679x; 1.2679x over previous
import functools

import jax
import jax.numpy as jnp
from jax import lax
from jax.experimental import pallas as pl
from jax.experimental.pallas import tpu as pltpu
from jax.experimental.pallas import tpu_sc as plsc

VOCAB = 1000000
DIM = 64
MAX_LEN = 200
B = 4096
EPS = 1e-5

NC = 2
NS = 16
NW = NC * NS

NVREG = DIM // 16

SENT_PER_W = B // NW
SC_CHUNK = 2
TOK_CHUNK = SC_CHUNK * MAX_LEN
N_CHUNKS = SENT_PER_W // SC_CHUNK
GATHER_SUB = 128


def _rsqrt_scalar(x):
    i = lax.bitcast_convert_type(x, jnp.int32)
    i = jnp.int32(0x5F3759DF) - lax.shift_right_arithmetic(i, 1)
    y = lax.bitcast_convert_type(i, jnp.float32)
    for _ in range(3):
        y = y * (jnp.float32(1.5) - jnp.float32(0.5) * x * y * y)
    return y


def _make_kernel():
    mesh = plsc.VectorSubcoreMesh(core_axis_name="c", subcore_axis_name="s")

    @functools.partial(
        pl.kernel,
        out_type=jax.ShapeDtypeStruct((B * MAX_LEN, DIM), jnp.float32),
        mesh=mesh,
        scratch_types=[
            pltpu.VMEM((TOK_CHUNK,), jnp.int32),
            pltpu.VMEM((TOK_CHUNK,), jnp.int32),
            pltpu.VMEM((TOK_CHUNK, DIM), jnp.float32),
            pltpu.VMEM((TOK_CHUNK, DIM), jnp.float32),
            pltpu.VMEM((TOK_CHUNK, DIM), jnp.float32),
            pltpu.VMEM((TOK_CHUNK, DIM), jnp.float32),
            pltpu.VMEM((MAX_LEN, DIM), jnp.float32),
            pltpu.VMEM((DIM,), jnp.float32),
            pltpu.VMEM((DIM,), jnp.float32),
            pltpu.SemaphoreType.DMA,
            pltpu.SemaphoreType.DMA,
            pltpu.SemaphoreType.DMA,
            pltpu.SemaphoreType.DMA,
        ],
        compiler_params=pltpu.CompilerParams(
            needs_layout_passes=False, use_tc_tiling_on_sc=False),
    )
    def emb_kernel(ids_hbm, table_hbm, pos_hbm, scale_hbm, bias_hbm,
                   out_hbm, idx0, idx1, rows0, rows1, ob0, ob1,
                   pos_v, scale_v, bias_v, sg0, sg1, so0, so1):
        wid = lax.axis_index("s") * NC + lax.axis_index("c")
        w_base = wid * (SENT_PER_W * MAX_LEN)

        idx_b = (idx0, idx1)
        rows_b = (rows0, rows1)
        out_b = (ob0, ob1)
        sg = (sg0, sg1)
        so = (so0, so1)

        pltpu.sync_copy(pos_hbm, pos_v)
        pltpu.sync_copy(scale_hbm, scale_v)
        pltpu.sync_copy(bias_hbm, bias_v)

        sc = [scale_v[pl.ds(k * 16, 16)] for k in range(NVREG)]
        bs = [bias_v[pl.ds(k * 16, 16)] for k in range(NVREG)]

        def stage(c, b):
            tok_base = w_base + c * TOK_CHUNK
            pltpu.sync_copy(ids_hbm.at[pl.ds(tok_base, TOK_CHUNK)], idx_b[b])
            for g in range(0, TOK_CHUNK, GATHER_SUB):
                n = min(GATHER_SUB, TOK_CHUNK - g)
                pltpu.async_copy(table_hbm.at[idx_b[b].at[pl.ds(g, n)]],
                                 rows_b[b].at[pl.ds(g, n)], sg[b])

        def wait_gather(b):
            for g in range(0, TOK_CHUNK, GATHER_SUB):
                n = min(GATHER_SUB, TOK_CHUNK - g)
                pltpu.make_async_copy(
                    table_hbm.at[idx_b[b].at[pl.ds(g, n)]],
                    rows_b[b].at[pl.ds(g, n)], sg[b]).wait()

        def fire_out(c, b):
            tok_base = w_base + c * TOK_CHUNK
            pltpu.async_copy(out_b[b],
                             out_hbm.at[pl.ds(tok_base, TOK_CHUNK)], so[b])

        def wait_out(c, b):
            tok_base = w_base + c * TOK_CHUNK
            pltpu.make_async_copy(
                out_b[b], out_hbm.at[pl.ds(tok_base, TOK_CHUNK)],
                so[b]).wait()

        def compute(b):
            rows_v = rows_b[b]
            out_v = out_b[b]

            @plsc.parallel_loop(0, MAX_LEN, unroll=2)
            def pos_body(p):
                pv = [pos_v[p, pl.ds(k * 16, 16)] for k in range(NVREG)]
                for s in range(SC_CHUNK):
                    t = s * MAX_LEN + p
                    h = [rows_v[t, pl.ds(k * 16, 16)] + pv[k]
                         for k in range(NVREG)]
                    ssum = (h[0] + h[1]) + (h[2] + h[3])
                    qsum = h[0] * h[0]
                    for k in range(1, NVREG):
                        qsum = h[k] * h[k] + qsum
                    tot_s = plsc.cumsum(ssum)[15]
                    tot_q = plsc.cumsum(qsum)[15]
                    mean = tot_s * jnp.float32(1.0 / DIM)
                    var = tot_q * jnp.float32(1.0 / DIM) - mean * mean
                    rstd = _rsqrt_scalar(var + jnp.float32(EPS))
                    a = rstd
                    nb = -mean * rstd
                    for k in range(NVREG):
                        out_v[t, pl.ds(k * 16, 16)] = (
                            (h[k] * a + nb) * sc[k] + bs[k])

        stage(0, 0)

        def pair_body(g, carry):
            c0 = g * 2
            c1 = c0 + 1
            stage(c1, 1)
            wait_gather(0)

            @pl.when(g > 0)
            def _():
                wait_out(c0 - 2, 0)

            compute(0)
            fire_out(c0, 0)

            @pl.when(g < N_CHUNKS // 2 - 1)
            def _():
                stage(c0 + 2, 0)

            wait_gather(1)

            @pl.when(g > 0)
            def _():
                wait_out(c1 - 2, 1)

            compute(1)
            fire_out(c1, 1)
            return carry

        lax.fori_loop(0, N_CHUNKS // 2, pair_body, 0)
        wait_out(N_CHUNKS - 2, 0)
        wait_out(N_CHUNKS - 1, 1)

    return emb_kernel


_EMB_KERNEL_CACHE = []


def kernel(input_ids, attention_mask, sentence_lengths, word_table,
           pos_table, ln_scale, ln_bias):
    del attention_mask, sentence_lengths
    if not _EMB_KERNEL_CACHE:
        _EMB_KERNEL_CACHE.append(_make_kernel())
    ids_flat = input_ids.reshape(B * MAX_LEN)
    out = _EMB_KERNEL_CACHE[0](ids_flat, word_table, pos_table, ln_scale,
                               ln_bias)
    return out.reshape(B, MAX_LEN, DIM)
